# baseline (speedup 1.0000x reference)
import functools

import jax
import jax.numpy as jnp
from jax import lax
from jax.experimental import pallas as pl
from jax.experimental.pallas import tpu as pltpu
from jax.experimental.pallas import tpu_sc as plsc

NC = 2
NS = 16
NW = NC * NS
B = 16384
D = 64
BPW = B // NW
CHUNK = 128
NCHUNK = BPW // CHUNK
LHALF = BPW // 2

_mesh = plsc.VectorSubcoreMesh(core_axis_name="c", subcore_axis_name="s")


@functools.partial(
    pl.kernel,
    out_type=(
        jax.ShapeDtypeStruct((B, D), jnp.float32),
        jax.ShapeDtypeStruct((B, D), jnp.float32),
        jax.ShapeDtypeStruct((B, 2 * D), jnp.float32),
    ),
    mesh=_mesh,
    compiler_params=pltpu.CompilerParams(use_tc_tiling_on_sc=False),
    scratch_types=[
        pltpu.VMEM((NCHUNK, CHUNK), jnp.int32),
        pltpu.VMEM((NCHUNK, CHUNK), jnp.int32),
        pltpu.VMEM((NCHUNK, CHUNK), jnp.int32),
        pltpu.VMEM((BPW, D), jnp.float32),
        pltpu.VMEM((BPW, D), jnp.float32),
        pltpu.VMEM((LHALF, 2 * D), jnp.float32),
        pltpu.SemaphoreType.DMA,
        pltpu.SemaphoreType.DMA,
        pltpu.SemaphoreType.DMA,
        pltpu.SemaphoreType.DMA,
        pltpu.SemaphoreType.DMA,
    ],
)
def _gather3(eu, et, el, iu, it, il, ou, ot, ol,
             idx_u, idx_t, idx_l, rows_u, rows_t, lines_v,
             sem_u, sem_t, sem_l, sem_w, sem_wl):
    wid = lax.axis_index("s") * NC + lax.axis_index("c")
    base = wid * BPW

    pltpu.sync_copy(iu.at[wid], idx_u)
    pltpu.sync_copy(it.at[wid], idx_t)
    pltpu.sync_copy(il.at[wid], idx_l)

    gu, gt, gl = [], [], []
    for j in range(NCHUNK):
        gu.append(pltpu.async_copy(
            eu.at[idx_u.at[j]], rows_u.at[pl.ds(j * CHUNK, CHUNK)], sem_u))
    for j in range(NCHUNK):
        gt.append(pltpu.async_copy(
            et.at[idx_t.at[j]], rows_t.at[pl.ds(j * CHUNK, CHUNK)], sem_t))
    for j in range(2):
        gl.append(pltpu.async_copy(
            el.at[idx_l.at[j]], lines_v.at[pl.ds(j * CHUNK, CHUNK)], sem_l))

    for c in gu:
        c.wait()
    wu = pltpu.async_copy(rows_u, ou.at[pl.ds(base, BPW)], sem_w)
    for c in gt:
        c.wait()
    wt = pltpu.async_copy(rows_t, ot.at[pl.ds(base, BPW)], sem_w)
    for c in gl:
        c.wait()
    wl = pltpu.async_copy(lines_v, ol.at[pl.ds(base, LHALF)], sem_wl)

    wl.wait()
    gl2 = []
    for j in range(2, 4):
        gl2.append(pltpu.async_copy(
            el.at[idx_l.at[j]],
            lines_v.at[pl.ds((j - 2) * CHUNK, CHUNK)], sem_l))
    for c in gl2:
        c.wait()
    wl2 = pltpu.async_copy(
        lines_v, ol.at[pl.ds(base + LHALF, LHALF)], sem_wl)
    wu.wait()
    wt.wait()
    wl2.wait()


def kernel(emb_user, emb_item, emb_tag, indices_user, indices_item, indices_tag):
    iu = indices_user.astype(jnp.int32)
    ii = indices_item.astype(jnp.int32)
    it = indices_tag.astype(jnp.int32)
    item_lines_tab = emb_item.reshape(500000, 2 * D)
    out_u, out_t, lines = _gather3(
        emb_user, emb_tag, item_lines_tab,
        iu.reshape(NW, NCHUNK, CHUNK),
        it.reshape(NW, NCHUNK, CHUNK),
        lax.shift_right_logical(ii, 1).reshape(NW, NCHUNK, CHUNK),
    )
    odd = (ii & 1)[:, None] == 1
    out_i = jnp.where(odd, lines[:, D:], lines[:, :D])
    return (out_u, out_i, out_t)
